# Initial kernel scaffold; baseline (speedup 1.0000x reference)
import jax
import jax.numpy as jnp
from jax.experimental import pallas as pl


def kernel(x, edge_index, batch, Wrel0, Wrel1, Wrel2, Wrel3, Wroot0, Wroot1, Wroot2, Wroot3, bc0, bc1, bc2, bc3, g0, g1, g2, g3, be0, be1, be2, be3, W1, b1, W2, b2):
    raise NotImplementedError("write your pallas kernel here")


# speedup vs baseline: 6.1437x; 6.1437x over previous
import functools

import jax
import jax.numpy as jnp
from jax import lax
from jax.experimental import pallas as pl
from jax.experimental.pallas import tpu as pltpu
from jax.experimental.pallas import tpu_sc as plsc

N = 10000
E = 320000
D = 128
C = 10
G = 128

NC = 2
NS = 16
NW = NC * NS

K = 80
EPT = E // NW
CH = EPT // K
NPAD = 10240
ZR = 80


def _scseg(h, src_e, dst_e):
    mesh = plsc.VectorSubcoreMesh(core_axis_name="c", subcore_axis_name="s")

    @functools.partial(
        pl.kernel,
        mesh=mesh,
        out_type=jax.ShapeDtypeStruct((2 * N, D), jnp.float32),
        scratch_types=[
            pltpu.VMEM((EPT,), jnp.int32),
            pltpu.VMEM((EPT,), jnp.int32),
            pltpu.VMEM((K,), jnp.int32),
            pltpu.VMEM((K,), jnp.int32),
            pltpu.VMEM((K, D), jnp.float32),
            pltpu.VMEM((ZR, D), jnp.float32),
            pltpu.VMEM_SHARED((NPAD, D), jnp.float32),
            pltpu.SemaphoreType.DMA,
        ],
    )
    def k(h_hbm, src_hbm, dst_hbm, out_hbm, src_all, dst_all, src_v, dst_v,
          rows_v, zbuf_v, acc_sh, sem):
        c = lax.axis_index("c")
        s = lax.axis_index("s")

        def zrow(i, carry):
            for j in range(D // 16):
                zbuf_v[i, pl.ds(j * 16, 16)] = jnp.zeros((16,), jnp.float32)
            return carry

        lax.fori_loop(0, ZR, zrow, 0)

        def zcopy(i, carry):
            pltpu.sync_copy(zbuf_v, acc_sh.at[pl.ds(s * 640 + i * ZR, ZR)])
            return carry

        lax.fori_loop(0, 640 // ZR, zcopy, 0)
        plsc.subcore_barrier()

        wid = c * NS + s
        pltpu.sync_copy(src_hbm.at[pl.ds(wid * EPT, EPT)], src_all)
        pltpu.sync_copy(dst_hbm.at[pl.ds(wid * EPT, EPT)], dst_all)

        def body(ch, carry):
            for j in range(K // 16):
                sl = pl.ds(ch * K + j * 16, 16)
                src_v[pl.ds(j * 16, 16)] = src_all[sl]
                dst_v[pl.ds(j * 16, 16)] = dst_all[sl]
            pltpu.async_copy(h_hbm.at[src_v], rows_v, sem).wait()
            pltpu.sync_copy(rows_v, acc_sh.at[dst_v], add=True)
            return carry

        lax.fori_loop(0, CH, body, 0)
        plsc.subcore_barrier()

        @pl.when(s < NS - 1)
        def _():
            pltpu.sync_copy(acc_sh.at[pl.ds(s * 640, 640)],
                            out_hbm.at[pl.ds(c * N + s * 640, 640)])

        @pl.when(s == NS - 1)
        def _():
            pltpu.sync_copy(acc_sh.at[pl.ds(9600, N - 9600)],
                            out_hbm.at[pl.ds(c * N + 9600, N - 9600)])

    return k(h, src_e, dst_e)


NB = 2000
NBLK = N // NB
_HI = jax.lax.Precision.HIGHEST
_CN = (((1,), (1,)), ((), ()))


def _dense(parts, h, Wrel, Wroot):
    def body(a0_ref, a1_ref, h_ref, wr_ref, wo_ref, x_ref, st_ref):
        i = pl.program_id(0)
        a = a0_ref[...] + a1_ref[...]
        xv = lax.dot_general(a, wr_ref[...], _CN, precision=_HI,
                             preferred_element_type=jnp.float32)
        xv = xv + lax.dot_general(h_ref[...], wo_ref[...], _CN, precision=_HI,
                                  preferred_element_type=jnp.float32)
        x_ref[...] = xv

        @pl.when(i == 0)
        def _():
            st_ref[...] = jnp.zeros((8, D), jnp.float32)

        st_ref[0:1, :] += jnp.sum(xv, axis=0, keepdims=True)
        st_ref[1:2, :] += jnp.sum(xv * xv, axis=0, keepdims=True)

        @pl.when(i == NBLK - 1)
        def _():
            mu = st_ref[0:1, :] / N
            var = st_ref[1:2, :] / N - mu * mu
            st_ref[0:1, :] = mu
            st_ref[1:2, :] = lax.rsqrt(var + 1e-5)

    return pl.pallas_call(
        body,
        grid=(NBLK,),
        in_specs=[
            pl.BlockSpec((NB, D), lambda i: (i, 0)),
            pl.BlockSpec((NB, D), lambda i: (i, 0)),
            pl.BlockSpec((NB, D), lambda i: (i, 0)),
            pl.BlockSpec((D, D), lambda i: (0, 0)),
            pl.BlockSpec((D, D), lambda i: (0, 0)),
        ],
        out_specs=[
            pl.BlockSpec((NB, D), lambda i: (i, 0)),
            pl.BlockSpec((8, D), lambda i: (0, 0)),
        ],
        out_shape=[
            jax.ShapeDtypeStruct((N, D), jnp.float32),
            jax.ShapeDtypeStruct((8, D), jnp.float32),
        ],
    )(parts[:N], parts[N:], h, Wrel, Wroot)


def _norm(X, st, g, be):
    def body(x_ref, st_ref, g_ref, be_ref, h_ref):
        mu = st_ref[0:1, :]
        rs = st_ref[1:2, :]
        h_ref[...] = jnp.maximum(
            (x_ref[...] - mu) * (rs * g_ref[...]) + be_ref[...], 0.0)

    return pl.pallas_call(
        body,
        grid=(NBLK,),
        in_specs=[
            pl.BlockSpec((NB, D), lambda i: (i, 0)),
            pl.BlockSpec((8, D), lambda i: (0, 0)),
            pl.BlockSpec((1, D), lambda i: (0, 0)),
            pl.BlockSpec((1, D), lambda i: (0, 0)),
        ],
        out_specs=pl.BlockSpec((NB, D), lambda i: (i, 0)),
        out_shape=jax.ShapeDtypeStruct((N, D), jnp.float32),
    )(X, st, g.reshape(1, D), be.reshape(1, D))


def _norm_pool(X, st, g, be, batch3d):
    def body(x_ref, st_ref, g_ref, be_ref, b_ref, p_ref):
        i = pl.program_id(0)
        mu = st_ref[0:1, :]
        rs = st_ref[1:2, :]
        h4 = jnp.maximum((x_ref[...] - mu) * (rs * g_ref[...]) + be_ref[...],
                         0.0)
        b = b_ref[...].reshape(1, NB)
        onehot = (b == lax.broadcasted_iota(jnp.int32, (G, 1), 0))

        @pl.when(i == 0)
        def _():
            p_ref[...] = jnp.zeros((G, D), jnp.float32)

        p_ref[...] += lax.dot_general(
            onehot.astype(jnp.float32), h4, (((1,), (0,)), ((), ())),
            precision=_HI, preferred_element_type=jnp.float32)

    return pl.pallas_call(
        body,
        grid=(NBLK,),
        in_specs=[
            pl.BlockSpec((NB, D), lambda i: (i, 0)),
            pl.BlockSpec((8, D), lambda i: (0, 0)),
            pl.BlockSpec((1, D), lambda i: (0, 0)),
            pl.BlockSpec((1, D), lambda i: (0, 0)),
            pl.BlockSpec((1, 1, NB), lambda i: (i, 0, 0)),
        ],
        out_specs=pl.BlockSpec((G, D), lambda i: (0, 0)),
        out_shape=jax.ShapeDtypeStruct((G, D), jnp.float32),
    )(X, st, g.reshape(1, D), be.reshape(1, D), batch3d)


def _mlp(pooled, W1, b1, W2, b2):
    def body(p_ref, w1_ref, b1_ref, w2_ref, b2_ref, o_ref):
        x1 = jnp.maximum(
            lax.dot_general(p_ref[...], w1_ref[...], _CN, precision=_HI,
                            preferred_element_type=jnp.float32) + b1_ref[...],
            0.0)
        o = lax.dot_general(x1, w2_ref[...], _CN, precision=_HI,
                            preferred_element_type=jnp.float32) + b2_ref[...]
        m = jnp.max(o, axis=1, keepdims=True)
        e = jnp.exp(o - m)
        lse = jnp.log(jnp.sum(e, axis=1, keepdims=True)) + m
        o_ref[...] = o - lse

    return pl.pallas_call(
        body,
        out_shape=jax.ShapeDtypeStruct((G, C), jnp.float32),
    )(pooled, W1, b1.reshape(1, D), W2, b2.reshape(1, C))


def kernel(x, edge_index, batch, Wrel0, Wrel1, Wrel2, Wrel3, Wroot0, Wroot1,
           Wroot2, Wroot3, bc0, bc1, bc2, bc3, g0, g1, g2, g3, be0, be1, be2,
           be3, W1, b1, W2, b2):
    src_e = edge_index[0]
    dst_e = edge_index[1]
    batch3d = batch.reshape(NBLK, 1, NB)
    Wrel = [Wrel0, Wrel1, Wrel2, Wrel3]
    Wroot = [Wroot0, Wroot1, Wroot2, Wroot3]
    gs = [g0, g1, g2, g3]
    bes = [be0, be1, be2, be3]
    h = x
    for i in range(3):
        parts = _scseg(h, src_e, dst_e)
        X, st = _dense(parts, h, Wrel[i], Wroot[i])
        h = _norm(X, st, gs[i], bes[i])
    parts = _scseg(h, src_e, dst_e)
    X, st = _dense(parts, h, Wrel[3], Wroot[3])
    pooled = _norm_pool(X, st, gs[3], bes[3], batch3d)
    return _mlp(pooled, W1, b1, W2, b2)


# speedup vs baseline: 9.7855x; 1.5928x over previous
import functools

import jax
import jax.numpy as jnp
from jax import lax
from jax.experimental import pallas as pl
from jax.experimental.pallas import tpu as pltpu
from jax.experimental.pallas import tpu_sc as plsc

N = 10000
E = 320000
D = 128
C = 10
G = 128

NC = 2
NS = 16
NW = NC * NS

K = 80
EPT = E // NW
CH = EPT // K
NPAD = 10240
ZR = 80


def _scseg(h, src_e, dst_e):
    mesh = plsc.VectorSubcoreMesh(core_axis_name="c", subcore_axis_name="s")

    @functools.partial(
        pl.kernel,
        mesh=mesh,
        out_type=jax.ShapeDtypeStruct((2 * N, D), jnp.float32),
        scratch_types=[
            pltpu.VMEM((EPT,), jnp.int32),
            pltpu.VMEM((EPT,), jnp.int32),
            pltpu.VMEM((K,), jnp.int32),
            pltpu.VMEM((K,), jnp.int32),
            pltpu.VMEM((K, D), jnp.float32),
            pltpu.VMEM((K, D), jnp.float32),
            pltpu.VMEM_SHARED((NPAD, D), jnp.float32),
            pltpu.SemaphoreType.DMA,
            pltpu.SemaphoreType.DMA,
        ],
    )
    def k(h_hbm, src_hbm, dst_hbm, out_hbm, src_all, dst_all, dst_v0, dst_v1,
          rows_v0, rows_v1, acc_sh, sem0, sem1):
        c = lax.axis_index("c")
        s = lax.axis_index("s")

        def zrow(i, carry):
            for j in range(D // 16):
                rows_v0[i, pl.ds(j * 16, 16)] = jnp.zeros((16,), jnp.float32)
            return carry

        lax.fori_loop(0, ZR, zrow, 0)

        def zcopy(i, carry):
            pltpu.sync_copy(rows_v0, acc_sh.at[pl.ds(s * 640 + i * ZR, ZR)])
            return carry

        lax.fori_loop(0, 640 // ZR, zcopy, 0)
        plsc.subcore_barrier()

        wid = c * NS + s
        pltpu.sync_copy(src_hbm.at[pl.ds(wid * EPT, EPT)], src_all)
        pltpu.sync_copy(dst_hbm.at[pl.ds(wid * EPT, EPT)], dst_all)

        def gather(ch, rows, sem):
            return pltpu.async_copy(
                h_hbm.at[src_all.at[pl.ds(ch * K, K)]], rows, sem)

        def gwait(ch, rows, sem):
            pltpu.make_async_copy(
                h_hbm.at[src_all.at[pl.ds(ch * K, K)]], rows, sem).wait()

        def scatter(ch, rows, dst_v):
            for j in range(K // 16):
                dst_v[pl.ds(j * 16, 16)] = dst_all[pl.ds(ch * K + j * 16, 16)]
            pltpu.sync_copy(rows, acc_sh.at[dst_v], add=True)

        gather(0, rows_v0, sem0)

        def body(t, carry):
            ch0 = 2 * t
            ch1 = 2 * t + 1

            @pl.when(ch1 < CH)
            def _():
                gather(ch1, rows_v1, sem1)

            gwait(ch0, rows_v0, sem0)
            scatter(ch0, rows_v0, dst_v0)

            @pl.when(ch0 + 2 < CH)
            def _():
                gather(ch0 + 2, rows_v0, sem0)

            @pl.when(ch1 < CH)
            def _():
                gwait(ch1, rows_v1, sem1)
                scatter(ch1, rows_v1, dst_v1)

            return carry

        lax.fori_loop(0, (CH + 1) // 2, body, 0)
        plsc.subcore_barrier()

        @pl.when(s < NS - 1)
        def _():
            pltpu.sync_copy(acc_sh.at[pl.ds(s * 640, 640)],
                            out_hbm.at[pl.ds(c * N + s * 640, 640)])

        @pl.when(s == NS - 1)
        def _():
            pltpu.sync_copy(acc_sh.at[pl.ds(9600, N - 9600)],
                            out_hbm.at[pl.ds(c * N + 9600, N - 9600)])

    return k(h, src_e, dst_e)


NB = 2000
NBLK = N // NB
_HI = jax.lax.Precision.HIGHEST
_CN = (((1,), (1,)), ((), ()))


def _dense(parts, h, Wrel, Wroot):
    def body(a0_ref, a1_ref, h_ref, wr_ref, wo_ref, x_ref, st_ref):
        i = pl.program_id(0)
        a = a0_ref[...] + a1_ref[...]
        xv = lax.dot_general(a, wr_ref[...], _CN, precision=_HI,
                             preferred_element_type=jnp.float32)
        xv = xv + lax.dot_general(h_ref[...], wo_ref[...], _CN, precision=_HI,
                                  preferred_element_type=jnp.float32)
        x_ref[...] = xv

        @pl.when(i == 0)
        def _():
            st_ref[...] = jnp.zeros((8, D), jnp.float32)

        st_ref[0:1, :] += jnp.sum(xv, axis=0, keepdims=True)
        st_ref[1:2, :] += jnp.sum(xv * xv, axis=0, keepdims=True)

        @pl.when(i == NBLK - 1)
        def _():
            mu = st_ref[0:1, :] / N
            var = st_ref[1:2, :] / N - mu * mu
            st_ref[0:1, :] = mu
            st_ref[1:2, :] = lax.rsqrt(var + 1e-5)

    return pl.pallas_call(
        body,
        grid=(NBLK,),
        in_specs=[
            pl.BlockSpec((NB, D), lambda i: (i, 0)),
            pl.BlockSpec((NB, D), lambda i: (i, 0)),
            pl.BlockSpec((NB, D), lambda i: (i, 0)),
            pl.BlockSpec((D, D), lambda i: (0, 0)),
            pl.BlockSpec((D, D), lambda i: (0, 0)),
        ],
        out_specs=[
            pl.BlockSpec((NB, D), lambda i: (i, 0)),
            pl.BlockSpec((8, D), lambda i: (0, 0)),
        ],
        out_shape=[
            jax.ShapeDtypeStruct((N, D), jnp.float32),
            jax.ShapeDtypeStruct((8, D), jnp.float32),
        ],
    )(parts[:N], parts[N:], h, Wrel, Wroot)


def _norm(X, st, g, be):
    def body(x_ref, st_ref, g_ref, be_ref, h_ref):
        mu = st_ref[0:1, :]
        rs = st_ref[1:2, :]
        h_ref[...] = jnp.maximum(
            (x_ref[...] - mu) * (rs * g_ref[...]) + be_ref[...], 0.0)

    return pl.pallas_call(
        body,
        grid=(NBLK,),
        in_specs=[
            pl.BlockSpec((NB, D), lambda i: (i, 0)),
            pl.BlockSpec((8, D), lambda i: (0, 0)),
            pl.BlockSpec((1, D), lambda i: (0, 0)),
            pl.BlockSpec((1, D), lambda i: (0, 0)),
        ],
        out_specs=pl.BlockSpec((NB, D), lambda i: (i, 0)),
        out_shape=jax.ShapeDtypeStruct((N, D), jnp.float32),
    )(X, st, g.reshape(1, D), be.reshape(1, D))


def _norm_pool(X, st, g, be, batch3d):
    def body(x_ref, st_ref, g_ref, be_ref, b_ref, p_ref):
        i = pl.program_id(0)
        mu = st_ref[0:1, :]
        rs = st_ref[1:2, :]
        h4 = jnp.maximum((x_ref[...] - mu) * (rs * g_ref[...]) + be_ref[...],
                         0.0)
        b = b_ref[...].reshape(1, NB)
        onehot = (b == lax.broadcasted_iota(jnp.int32, (G, 1), 0))

        @pl.when(i == 0)
        def _():
            p_ref[...] = jnp.zeros((G, D), jnp.float32)

        p_ref[...] += lax.dot_general(
            onehot.astype(jnp.float32), h4, (((1,), (0,)), ((), ())),
            precision=_HI, preferred_element_type=jnp.float32)

    return pl.pallas_call(
        body,
        grid=(NBLK,),
        in_specs=[
            pl.BlockSpec((NB, D), lambda i: (i, 0)),
            pl.BlockSpec((8, D), lambda i: (0, 0)),
            pl.BlockSpec((1, D), lambda i: (0, 0)),
            pl.BlockSpec((1, D), lambda i: (0, 0)),
            pl.BlockSpec((1, 1, NB), lambda i: (i, 0, 0)),
        ],
        out_specs=pl.BlockSpec((G, D), lambda i: (0, 0)),
        out_shape=jax.ShapeDtypeStruct((G, D), jnp.float32),
    )(X, st, g.reshape(1, D), be.reshape(1, D), batch3d)


def _mlp(pooled, W1, b1, W2, b2):
    def body(p_ref, w1_ref, b1_ref, w2_ref, b2_ref, o_ref):
        x1 = jnp.maximum(
            lax.dot_general(p_ref[...], w1_ref[...], _CN, precision=_HI,
                            preferred_element_type=jnp.float32) + b1_ref[...],
            0.0)
        o = lax.dot_general(x1, w2_ref[...], _CN, precision=_HI,
                            preferred_element_type=jnp.float32) + b2_ref[...]
        m = jnp.max(o, axis=1, keepdims=True)
        e = jnp.exp(o - m)
        lse = jnp.log(jnp.sum(e, axis=1, keepdims=True)) + m
        o_ref[...] = o - lse

    return pl.pallas_call(
        body,
        out_shape=jax.ShapeDtypeStruct((G, C), jnp.float32),
    )(pooled, W1, b1.reshape(1, D), W2, b2.reshape(1, C))


def kernel(x, edge_index, batch, Wrel0, Wrel1, Wrel2, Wrel3, Wroot0, Wroot1,
           Wroot2, Wroot3, bc0, bc1, bc2, bc3, g0, g1, g2, g3, be0, be1, be2,
           be3, W1, b1, W2, b2):
    src_e = edge_index[0]
    dst_e = edge_index[1]
    batch3d = batch.reshape(NBLK, 1, NB)
    Wrel = [Wrel0, Wrel1, Wrel2, Wrel3]
    Wroot = [Wroot0, Wroot1, Wroot2, Wroot3]
    gs = [g0, g1, g2, g3]
    bes = [be0, be1, be2, be3]
    h = x
    for i in range(3):
        parts = _scseg(h, src_e, dst_e)
        X, st = _dense(parts, h, Wrel[i], Wroot[i])
        h = _norm(X, st, gs[i], bes[i])
    parts = _scseg(h, src_e, dst_e)
    X, st = _dense(parts, h, Wrel[3], Wroot[3])
    pooled = _norm_pool(X, st, gs[3], bes[3], batch3d)
    return _mlp(pooled, W1, b1, W2, b2)
